# Initial kernel scaffold; baseline (speedup 1.0000x reference)
import jax
import jax.numpy as jnp
from jax.experimental import pallas as pl


def kernel(y, perm, coeffs):
    raise NotImplementedError("write your pallas kernel here")


# speedup vs baseline: 2.3406x; 2.3406x over previous
import functools

import jax
import jax.numpy as jnp
from jax import lax
from jax.experimental import pallas as pl
from jax.experimental.pallas import tpu as pltpu
from jax.experimental.pallas import tpu_sc as plsc

_B = 16384
_C = 250
_NC = 2
_NS = 16
_NW = _NC * _NS
_RW = _B // _NW
_RB = 64
_NCH = _RW // _RB
_BUF = _RB * _C
_L = 16


def _mix_body(y_hbm, perm_hbm, c_hbm, out_hbm, y_v, p_v, c_v, buf):
    wid = lax.axis_index("s") * _NC + lax.axis_index("c")
    base = wid * _RW

    pltpu.sync_copy(y_hbm, y_v)
    pltpu.sync_copy(perm_hbm.at[pl.ds(base, _RW)], p_v)
    pltpu.sync_copy(c_hbm.at[pl.ds(base, _RW)], c_v)

    zero16 = jnp.zeros((_L,), jnp.float32)
    iota = lax.iota(jnp.int32, _L)

    def zinit(i, carry):
        buf[pl.ds(i * _L, _L)] = zero16
        return carry

    lax.fori_loop(0, _BUF // _L, zinit, 0)

    def chunk(ck, carry):
        def grp(g, carry2):
            roff = ck * _RB + g * _L
            yv = y_v[pl.ds(base + roff, _L)]
            pv = p_v[pl.ds(roff, _L)]
            yp = plsc.load_gather(y_v, [pv])
            cv = c_v[pl.ds(roff, _L)]
            lidx = (g * _L + iota) * _C
            plsc.store_scatter(buf, [lidx + yv], cv)
            plsc.addupdate_scatter(buf, [lidx + yp], 1.0 - cv)
            return carry2

        lax.fori_loop(0, _RB // _L, grp, 0)
        pltpu.sync_copy(buf, out_hbm.at[pl.ds((base + ck * _RB) * _C, _BUF)])

        def zgrp(g, carry2):
            roff = ck * _RB + g * _L
            yv = y_v[pl.ds(base + roff, _L)]
            pv = p_v[pl.ds(roff, _L)]
            yp = plsc.load_gather(y_v, [pv])
            lidx = (g * _L + iota) * _C
            plsc.store_scatter(buf, [lidx + yv], zero16)
            plsc.store_scatter(buf, [lidx + yp], zero16)
            return carry2

        lax.fori_loop(0, _RB // _L, zgrp, 0)
        return carry

    lax.fori_loop(0, _NCH, chunk, 0)


@functools.partial(jax.jit)
def kernel(y, perm, coeffs):
    mesh = plsc.VectorSubcoreMesh(core_axis_name="c", subcore_axis_name="s")
    mix = pl.kernel(
        _mix_body,
        mesh=mesh,
        out_type=jax.ShapeDtypeStruct((_B * _C,), jnp.float32),
        compiler_params=pltpu.CompilerParams(needs_layout_passes=False),
        scratch_types=[
            pltpu.VMEM((_B,), jnp.int32),
            pltpu.VMEM((_RW,), jnp.int32),
            pltpu.VMEM((_RW,), jnp.float32),
            pltpu.VMEM((_BUF,), jnp.float32),
        ],
    )
    flat = mix(y.astype(jnp.int32), perm.astype(jnp.int32), coeffs)
    return (perm, coeffs, flat.reshape(_B, _C))


# speedup vs baseline: 4.5413x; 1.9402x over previous
import functools

import jax
import jax.numpy as jnp
from jax import lax
from jax.experimental import pallas as pl
from jax.experimental.pallas import tpu as pltpu
from jax.experimental.pallas import tpu_sc as plsc

_B = 16384
_C = 250
_NC = 2
_NS = 16
_NW = _NC * _NS
_RW = _B // _NW
_RB = 64
_NCH = _RW // _RB
_L = 16


def _mix_body(y_hbm, perm_hbm, c_hbm, out_hbm, y_v, p_v, c_v, buf):
    wid = lax.axis_index("s") * _NC + lax.axis_index("c")
    base = wid * _RW

    pltpu.sync_copy(y_hbm, y_v)
    pltpu.sync_copy(perm_hbm.at[pl.ds(base, _RW)], p_v)
    pltpu.sync_copy(c_hbm.at[pl.ds(base, _RW)], c_v)

    zero16 = jnp.zeros((_L,), jnp.float32)
    iota = lax.iota(jnp.int32, _L)
    tail_mask = iota < jnp.int32(_C % _L)

    def zinit(r, carry):
        for j in range(_C // _L):
            buf[r, pl.ds(j * _L, _L)] = zero16
        rvec = jnp.full((_L,), 0, jnp.int32) + r
        plsc.store_scatter(buf, [rvec, (_C // _L) * _L + iota], zero16,
                           mask=tail_mask)
        return carry

    lax.fori_loop(0, _RB, zinit, 0)

    def chunk(ck, carry):
        def grp(g, carry2):
            roff = ck * _RB + g * _L
            yv = y_v[pl.ds(base + roff, _L)]
            pv = p_v[pl.ds(roff, _L)]
            yp = plsc.load_gather(y_v, [pv])
            cv = c_v[pl.ds(roff, _L)]
            rvec = g * _L + iota
            plsc.store_scatter(buf, [rvec, yv], cv)
            plsc.addupdate_scatter(buf, [rvec, yp], 1.0 - cv)
            return carry2

        lax.fori_loop(0, _RB // _L, grp, 0)
        pltpu.sync_copy(buf, out_hbm.at[pl.ds(base + ck * _RB, _RB)])

        def zgrp(g, carry2):
            roff = ck * _RB + g * _L
            yv = y_v[pl.ds(base + roff, _L)]
            pv = p_v[pl.ds(roff, _L)]
            yp = plsc.load_gather(y_v, [pv])
            rvec = g * _L + iota
            plsc.store_scatter(buf, [rvec, yv], zero16)
            plsc.store_scatter(buf, [rvec, yp], zero16)
            return carry2

        lax.fori_loop(0, _RB // _L, zgrp, 0)
        return carry

    lax.fori_loop(0, _NCH, chunk, 0)


@functools.partial(jax.jit)
def kernel(y, perm, coeffs):
    mesh = plsc.VectorSubcoreMesh(core_axis_name="c", subcore_axis_name="s")
    mix = pl.kernel(
        _mix_body,
        mesh=mesh,
        out_type=jax.ShapeDtypeStruct((_B, _C), jnp.float32),
        compiler_params=pltpu.CompilerParams(needs_layout_passes=False),
        scratch_types=[
            pltpu.VMEM((_B,), jnp.int32),
            pltpu.VMEM((_RW,), jnp.int32),
            pltpu.VMEM((_RW,), jnp.float32),
            pltpu.VMEM((_RB, _C), jnp.float32),
        ],
    )
    y_mixed = mix(y.astype(jnp.int32), perm.astype(jnp.int32), coeffs)
    return (perm, coeffs, y_mixed)
